# baseline (speedup 1.0000x reference)
import jax
import jax.numpy as jnp
from jax import lax
from jax.experimental import pallas as pl
from jax.experimental.pallas import tpu as pltpu
from jax.experimental.pallas import tpu_sc as plsc

D_MODEL = 2048
B_TOTAL = 4 * 2048

_info = plsc.get_sparse_core_info()
NC, NS = _info.num_cores, _info.num_subcores
NW = NC * NS
B_PER_W = B_TOTAL // NW
CHUNK = 8
N_CHUNKS = B_PER_W // CHUNK
NBUF = 7
LOOKAHEAD = 5
N_MAIN = (N_CHUNKS // NBUF) * NBUF


def _embed_body(ids_hbm, table_hbm, out_hbm, idx_v, rows_v, gsems, ssems):
    wid = lax.axis_index("s") * NC + lax.axis_index("c")
    base = wid * B_PER_W
    pltpu.sync_copy(ids_hbm.at[pl.ds(base, B_PER_W)], idx_v)

    def issue_gather(j, b):
        pltpu.async_copy(
            table_hbm.at[idx_v.at[pl.ds(j * CHUNK, CHUNK)]],
            rows_v.at[b],
            gsems.at[b],
        )

    def wait_gather(b):
        pltpu.make_async_copy(
            table_hbm.at[pl.ds(0, CHUNK)], rows_v.at[b], gsems.at[b]
        ).wait()

    def issue_store(j, b):
        pltpu.async_copy(
            rows_v.at[b], out_hbm.at[pl.ds(base + j * CHUNK, CHUNK)], ssems.at[b]
        )

    def wait_store(b):
        pltpu.make_async_copy(
            rows_v.at[b], out_hbm.at[pl.ds(0, CHUNK)], ssems.at[b]
        ).wait()

    for j0 in range(LOOKAHEAD):
        issue_gather(j0, j0)

    @pl.loop(0, N_MAIN, step=NBUF)
    def _(j_base):
        for b in range(NBUF):
            j = j_base + b
            wait_gather(b)
            issue_store(j, b)
            jn = j + LOOKAHEAD
            bn = (b + LOOKAHEAD) % NBUF

            @pl.when(jn < N_CHUNKS)
            def _():
                @pl.when(jn >= NBUF)
                def _():
                    wait_store(bn)

                issue_gather(jn, bn)

    for j in range(N_MAIN, N_CHUNKS):
        b = j % NBUF
        wait_gather(b)
        issue_store(j, b)
    for b in range(NBUF):
        wait_store(b)


@jax.jit
def _embed(ids_flat, table):
    mesh = plsc.VectorSubcoreMesh(core_axis_name="c", subcore_axis_name="s")
    return pl.kernel(
        _embed_body,
        out_type=jax.ShapeDtypeStruct((B_TOTAL, D_MODEL), jnp.float32),
        mesh=mesh,
        compiler_params=pltpu.CompilerParams(
            disable_bounds_checks=True,
            disable_semaphore_checks=True,
        ),
        scratch_types=[
            pltpu.VMEM((B_PER_W,), jnp.int32),
            pltpu.VMEM((NBUF, CHUNK, D_MODEL), jnp.float32),
            pltpu.SemaphoreType.DMA((NBUF,)),
            pltpu.SemaphoreType.DMA((NBUF,)),
        ],
    )(ids_flat, table)


def kernel(input_ids, attention_mask, position_ids, embed_weight):
    ids_flat = input_ids.reshape(-1).astype(jnp.int32)
    out = _embed(ids_flat, embed_weight)
    inputs_embeds = out.reshape(input_ids.shape[0], input_ids.shape[1], D_MODEL)
    return (inputs_embeds, attention_mask, position_ids)


# speedup vs baseline: 1.0043x; 1.0043x over previous
import jax
import jax.numpy as jnp
from jax import lax
from jax.experimental import pallas as pl
from jax.experimental.pallas import tpu as pltpu
from jax.experimental.pallas import tpu_sc as plsc

D_MODEL = 2048
B_TOTAL = 4 * 2048

_info = plsc.get_sparse_core_info()
NC, NS = _info.num_cores, _info.num_subcores
NW = NC * NS
B_PER_W = B_TOTAL // NW
CHUNK = 8
N_CHUNKS = B_PER_W // CHUNK
NBUF = 7
LOOKAHEAD = 5


def _embed_body(ids_hbm, table_hbm, out_hbm, idx_v, rows_v, gsems, ssems):
    wid = lax.axis_index("s") * NC + lax.axis_index("c")
    base = wid * B_PER_W
    pltpu.sync_copy(ids_hbm.at[pl.ds(base, B_PER_W)], idx_v)

    @pl.loop(0, N_CHUNKS + LOOKAHEAD)
    def _(t):
        jn = t
        j = t - LOOKAHEAD

        @pl.when(jn < N_CHUNKS)
        def _():
            bn = lax.rem(jn, NBUF)

            @pl.when(jn >= NBUF)
            def _():
                pltpu.make_async_copy(
                    rows_v.at[bn], out_hbm.at[pl.ds(0, CHUNK)], ssems.at[bn]
                ).wait()

            pltpu.async_copy(
                table_hbm.at[idx_v.at[pl.ds(jn * CHUNK, CHUNK)]],
                rows_v.at[bn],
                gsems.at[bn],
            )

        @pl.when(j >= 0)
        def _():
            b = lax.rem(j, NBUF)
            pltpu.make_async_copy(
                table_hbm.at[pl.ds(0, CHUNK)], rows_v.at[b], gsems.at[b]
            ).wait()
            pltpu.async_copy(
                rows_v.at[b], out_hbm.at[pl.ds(base + j * CHUNK, CHUNK)], ssems.at[b]
            )

    @pl.loop(0, NBUF)
    def _(b):
        pltpu.make_async_copy(
            rows_v.at[b], out_hbm.at[pl.ds(0, CHUNK)], ssems.at[b]
        ).wait()


@jax.jit
def _embed(ids_flat, table):
    mesh = plsc.VectorSubcoreMesh(core_axis_name="c", subcore_axis_name="s")
    return pl.kernel(
        _embed_body,
        out_type=jax.ShapeDtypeStruct((B_TOTAL, D_MODEL), jnp.float32),
        mesh=mesh,
        compiler_params=pltpu.CompilerParams(
            disable_bounds_checks=True,
            disable_semaphore_checks=True,
        ),
        scratch_types=[
            pltpu.VMEM((B_PER_W,), jnp.int32),
            pltpu.VMEM((NBUF, CHUNK, D_MODEL), jnp.float32),
            pltpu.SemaphoreType.DMA((NBUF,)),
            pltpu.SemaphoreType.DMA((NBUF,)),
        ],
    )(ids_flat, table)


def kernel(input_ids, attention_mask, position_ids, embed_weight):
    ids_flat = input_ids.reshape(-1).astype(jnp.int32)
    out = _embed(ids_flat, embed_weight)
    inputs_embeds = out.reshape(input_ids.shape[0], input_ids.shape[1], D_MODEL)
    return (inputs_embeds, attention_mask, position_ids)


# speedup vs baseline: 1.0087x; 1.0044x over previous
import jax
import jax.numpy as jnp
from jax import lax
from jax.experimental import pallas as pl
from jax.experimental.pallas import tpu as pltpu
from jax.experimental.pallas import tpu_sc as plsc

D_MODEL = 2048
B_TOTAL = 4 * 2048

_info = plsc.get_sparse_core_info()
NC, NS = _info.num_cores, _info.num_subcores
NW = NC * NS
B_PER_W = B_TOTAL // NW
CHUNK = 8
N_CHUNKS = B_PER_W // CHUNK
NBUF = 7
LOOKAHEAD = 4


def _embed_body(ids_hbm, table_hbm, out_hbm, idx_v, rows_v, gsems, ssems):
    wid = lax.axis_index("s") * NC + lax.axis_index("c")
    base = wid * B_PER_W
    pltpu.sync_copy(ids_hbm.at[pl.ds(base, B_PER_W)], idx_v)

    @pl.loop(0, N_CHUNKS + LOOKAHEAD)
    def _(t):
        jn = t
        j = t - LOOKAHEAD

        @pl.when(jn < N_CHUNKS)
        def _():
            bn = lax.rem(jn, NBUF)

            @pl.when(jn >= NBUF)
            def _():
                pltpu.make_async_copy(
                    rows_v.at[bn], out_hbm.at[pl.ds(0, CHUNK)], ssems.at[bn]
                ).wait()

            pltpu.async_copy(
                table_hbm.at[idx_v.at[pl.ds(jn * CHUNK, CHUNK)]],
                rows_v.at[bn],
                gsems.at[bn],
            )

        @pl.when(j >= 0)
        def _():
            b = lax.rem(j, NBUF)
            pltpu.make_async_copy(
                table_hbm.at[pl.ds(0, CHUNK)], rows_v.at[b], gsems.at[b]
            ).wait()
            pltpu.async_copy(
                rows_v.at[b], out_hbm.at[pl.ds(base + j * CHUNK, CHUNK)], ssems.at[b]
            )

    @pl.loop(0, NBUF)
    def _(b):
        pltpu.make_async_copy(
            rows_v.at[b], out_hbm.at[pl.ds(0, CHUNK)], ssems.at[b]
        ).wait()


@jax.jit
def _embed(ids_flat, table):
    mesh = plsc.VectorSubcoreMesh(core_axis_name="c", subcore_axis_name="s")
    return pl.kernel(
        _embed_body,
        out_type=jax.ShapeDtypeStruct((B_TOTAL, D_MODEL), jnp.float32),
        mesh=mesh,
        compiler_params=pltpu.CompilerParams(
            disable_bounds_checks=True,
            disable_semaphore_checks=True,
        ),
        scratch_types=[
            pltpu.VMEM((B_PER_W,), jnp.int32),
            pltpu.VMEM((NBUF, CHUNK, D_MODEL), jnp.float32),
            pltpu.SemaphoreType.DMA((NBUF,)),
            pltpu.SemaphoreType.DMA((NBUF,)),
        ],
    )(ids_flat, table)


def kernel(input_ids, attention_mask, position_ids, embed_weight):
    ids_flat = input_ids.reshape(-1).astype(jnp.int32)
    out = _embed(ids_flat, embed_weight)
    inputs_embeds = out.reshape(input_ids.shape[0], input_ids.shape[1], D_MODEL)
    return (inputs_embeds, attention_mask, position_ids)


# speedup vs baseline: 1.0102x; 1.0014x over previous
import jax
import jax.numpy as jnp
from jax import lax
from jax.experimental import pallas as pl
from jax.experimental.pallas import tpu as pltpu
from jax.experimental.pallas import tpu_sc as plsc

D_MODEL = 2048
B_TOTAL = 4 * 2048

_info = plsc.get_sparse_core_info()
NC, NS = _info.num_cores, _info.num_subcores
NW = NC * NS
B_PER_W = B_TOTAL // NW
CHUNK = 8
N_CHUNKS = B_PER_W // CHUNK
NBUF = 7
LOOKAHEAD = 3


def _embed_body(ids_hbm, table_hbm, out_hbm, idx_v, rows_v, gsems, ssems):
    wid = lax.axis_index("s") * NC + lax.axis_index("c")
    base = wid * B_PER_W
    pltpu.sync_copy(ids_hbm.at[pl.ds(base, B_PER_W)], idx_v)

    @pl.loop(0, N_CHUNKS + LOOKAHEAD)
    def _(t):
        jn = t
        j = t - LOOKAHEAD

        @pl.when(jn < N_CHUNKS)
        def _():
            bn = lax.rem(jn, NBUF)

            @pl.when(jn >= NBUF)
            def _():
                pltpu.make_async_copy(
                    rows_v.at[bn], out_hbm.at[pl.ds(0, CHUNK)], ssems.at[bn]
                ).wait()

            pltpu.async_copy(
                table_hbm.at[idx_v.at[pl.ds(jn * CHUNK, CHUNK)]],
                rows_v.at[bn],
                gsems.at[bn],
            )

        @pl.when(j >= 0)
        def _():
            b = lax.rem(j, NBUF)
            pltpu.make_async_copy(
                table_hbm.at[pl.ds(0, CHUNK)], rows_v.at[b], gsems.at[b]
            ).wait()
            pltpu.async_copy(
                rows_v.at[b], out_hbm.at[pl.ds(base + j * CHUNK, CHUNK)], ssems.at[b]
            )

    @pl.loop(0, NBUF)
    def _(b):
        pltpu.make_async_copy(
            rows_v.at[b], out_hbm.at[pl.ds(0, CHUNK)], ssems.at[b]
        ).wait()


@jax.jit
def _embed(ids_flat, table):
    mesh = plsc.VectorSubcoreMesh(core_axis_name="c", subcore_axis_name="s")
    return pl.kernel(
        _embed_body,
        out_type=jax.ShapeDtypeStruct((B_TOTAL, D_MODEL), jnp.float32),
        mesh=mesh,
        compiler_params=pltpu.CompilerParams(
            disable_bounds_checks=True,
            disable_semaphore_checks=True,
        ),
        scratch_types=[
            pltpu.VMEM((B_PER_W,), jnp.int32),
            pltpu.VMEM((NBUF, CHUNK, D_MODEL), jnp.float32),
            pltpu.SemaphoreType.DMA((NBUF,)),
            pltpu.SemaphoreType.DMA((NBUF,)),
        ],
    )(ids_flat, table)


def kernel(input_ids, attention_mask, position_ids, embed_weight):
    ids_flat = input_ids.reshape(-1).astype(jnp.int32)
    out = _embed(ids_flat, embed_weight)
    inputs_embeds = out.reshape(input_ids.shape[0], input_ids.shape[1], D_MODEL)
    return (inputs_embeds, attention_mask, position_ids)
